# baseline (speedup 1.0000x reference)
import functools

import jax
import jax.numpy as jnp
from jax import lax
from jax.experimental import pallas as pl
from jax.experimental.pallas import tpu as pltpu
from jax.experimental.pallas import tpu_sc as plsc

_B, _T, _D = 16, 512, 384
_L = 2048
_FCH = 64
_NC = (_L // 2) // _FCH
_SPN = 80
_SPAD = _L + 16


def _make_expand():
    mesh = plsc.VectorSubcoreMesh(core_axis_name="c", subcore_axis_name="s")

    @functools.partial(
        pl.kernel,
        mesh=mesh,
        compiler_params=pltpu.CompilerParams(needs_layout_passes=False),
        out_type=[
            jax.ShapeDtypeStruct((_B * _L, _D), jnp.float32),
            jax.ShapeDtypeStruct((_B, 16), jnp.int32),
        ],
        scratch_types=[
            pltpu.VMEM((_T,), jnp.float32),
            pltpu.VMEM((_SPAD,), jnp.int32),
            pltpu.VMEM((_L // 2 + _FCH,), jnp.int32),
            pltpu.VMEM((2, _SPN, _D), jnp.float32),
            pltpu.VMEM((2, _FCH, _D), jnp.float32),
            pltpu.VMEM((16,), jnp.int32),
            pltpu.VMEM((16,), jnp.int32),
            pltpu.SemaphoreType.DMA,
            pltpu.SemaphoreType.DMA,
            pltpu.SemaphoreType.DMA,
            pltpu.SemaphoreType.DMA,
        ],
    )
    def expand(table_hbm, dur_hbm, ml_hbm, out_hbm, len_hbm,
               dur_v, s_v, idx_v, span_v, outb_v, len_v, ml_v,
               gsem_a, gsem_b, ssem_a, ssem_b):
        c = lax.axis_index("c")
        s = lax.axis_index("s")
        b = c * 8 + s // 2
        h = s % 2

        pltpu.sync_copy(dur_hbm.at[b], dur_v)
        pltpu.sync_copy(ml_hbm, ml_v)

        lanes = jnp.arange(16, dtype=jnp.int32)
        neg16 = jnp.full((16,), -1, jnp.int32)

        def init_s(i, carry):
            s_v[pl.ds(i * 16, 16)] = neg16
            return carry

        lax.fori_loop(0, _SPAD // 16, init_s, 0)

        win_base = h * (_L // 2)

        def phase_a(carries, i):
            carry_len, carry_max = carries
            d = jnp.maximum(dur_v[pl.ds(i * 16, 16)], 0.0).astype(jnp.int32)
            ends = plsc.cumsum(d) + carry_len
            starts = ends - d
            tid = i * 16 + lanes
            m = (d > 0) & (starts < _L)
            starts_c = jnp.minimum(starts, _SPAD - 16)
            plsc.store_scatter(s_v, [starts_c], tid, mask=m)
            cmx = jnp.max(jnp.where(m & (starts < win_base), tid, -1))
            return (jnp.max(ends), jnp.maximum(carry_max, cmx))

        def phase_a_body(i, carries):
            return phase_a(carries, i)

        length, carry0 = lax.fori_loop(
            0, _T // 16, phase_a_body, (jnp.int32(0), jnp.int32(-1))
        )

        len_v[...] = jnp.broadcast_to(length, (16,))

        @pl.when(h == 0)
        def _():
            pltpu.sync_copy(len_v, len_hbm.at[b])

        base = b * _T
        win0 = h * (_L // 2 // 16)

        def phase_b(i, carry):
            v = s_v[pl.ds((win0 + i) * 16, 16)]
            cm = jnp.maximum(plsc.cummax(v), carry)
            g = base + jnp.minimum(jnp.maximum(cm, 0), _T - 1)
            idx_v[pl.ds(i * 16, 16)] = g
            return jnp.max(cm)

        cm_fin = lax.fori_loop(0, _L // 2 // 16, phase_b, carry0)

        g_last = jnp.broadcast_to(
            base + jnp.minimum(jnp.maximum(cm_fin, 0), _T - 1), (16,)
        ).astype(jnp.int32)
        for q in range(_FCH // 16):
            idx_v[pl.ds(_L // 2 + q * 16, 16)] = g_last

        mls = jnp.max(ml_v[...])
        len_eff = jnp.minimum(length, mls)
        out_base = b * _L + win_base
        cap = _B * _T - _SPN

        def chunk_info(k):
            first_g = idx_v[pl.ds(k * _FCH, 16)]
            last_g = idx_v[pl.ds(k * _FCH + _FCH - 16, 16)]
            row0 = jnp.minimum((first_g[0] // 8) * 8, cap)
            ok = (last_g[15] - row0) <= (_SPN - 1)
            return row0, ok

        def read_issue(k, par, row0, ok, sem):
            @pl.when(ok)
            def _():
                pltpu.async_copy(
                    table_hbm.at[pl.ds(row0, _SPN)], span_v.at[par], sem
                )

            @pl.when(jnp.logical_not(ok))
            def _():
                pltpu.async_copy(
                    table_hbm.at[idx_v.at[pl.ds(k * _FCH, _SPN)]],
                    span_v.at[par], sem,
                )

        def read_wait(par, sem):
            pltpu.make_async_copy(
                table_hbm.at[pl.ds(0, _SPN)], span_v.at[par], sem
            ).wait()

        def scat_issue(k, par, sem):
            pltpu.async_copy(
                outb_v.at[par],
                out_hbm.at[pl.ds(out_base + k * _FCH, _FCH)], sem,
            )

        def scat_drain(par, sem):
            pltpu.make_async_copy(
                outb_v.at[par], out_hbm.at[pl.ds(out_base, _FCH)], sem
            ).wait()

        def expand_chunk(k, par, row0, ok):
            span_ref = span_v.at[par]
            outb_ref = outb_v.at[par]
            fbase = win_base + k * _FCH

            def group(gi, carry):
                idxg = idx_v[pl.ds(k * _FCH + gi * 16, 16)]
                loff = jnp.where(ok, idxg - row0, gi * 16 + lanes)
                for lane in range(16):
                    lr = loff[lane]
                    vf = jnp.where(
                        fbase + gi * 16 + lane < len_eff,
                        jnp.float32(1.0), jnp.float32(0.0),
                    )
                    row = gi * 16 + lane
                    for jj in range(_D // 16):
                        outb_ref[row, pl.ds(jj * 16, 16)] = (
                            span_ref[lr, pl.ds(jj * 16, 16)] * vf
                        )
                return carry

            lax.fori_loop(0, _FCH // 16, group, 0)

        row0_0, ok_0 = chunk_info(jnp.int32(0))
        read_issue(jnp.int32(0), 0, row0_0, ok_0, gsem_a)

        def outer(j, carry):
            row0_e, ok_e = carry
            k0 = 2 * j
            k1 = 2 * j + 1

            @pl.when(j >= 1)
            def _():
                scat_drain(0, ssem_a)

            row0_o, ok_o = chunk_info(k1)
            read_issue(k1, 1, row0_o, ok_o, gsem_b)

            read_wait(0, gsem_a)
            expand_chunk(k0, 0, row0_e, ok_e)
            scat_issue(k0, 0, ssem_a)

            @pl.when(j >= 1)
            def _():
                scat_drain(1, ssem_b)

            knext = jnp.minimum(k0 + 2, _NC - 1)
            row0_n, ok_n = chunk_info(knext)

            @pl.when(j < _NC // 2 - 1)
            def _():
                read_issue(knext, 0, row0_n, ok_n, gsem_a)

            read_wait(1, gsem_b)
            expand_chunk(k1, 1, row0_o, ok_o)
            scat_issue(k1, 1, ssem_b)
            return (row0_n, ok_n)

        lax.fori_loop(0, _NC // 2, outer, (row0_0, ok_0))
        scat_drain(0, ssem_a)
        scat_drain(1, ssem_b)

    return expand


_EXPAND = _make_expand()


def kernel(hidden, durations, max_len):
    B, T, D = hidden.shape
    table = hidden.reshape(B * T, D)
    ml = jnp.minimum(jnp.asarray(max_len, jnp.int32), _L)
    mlv = jnp.broadcast_to(ml, (16,))
    out2d, len2d = _EXPAND(table, durations, mlv)
    return out2d.reshape(B, _L, D), len2d[:, 0]


# speedup vs baseline: 1.2263x; 1.2263x over previous
import functools

import jax
import jax.numpy as jnp
from jax import lax
from jax.experimental import pallas as pl
from jax.experimental.pallas import tpu as pltpu
from jax.experimental.pallas import tpu_sc as plsc

_B, _T, _D = 16, 512, 384
_L = 2048
_FCH = 64
_NC = (_L // 2) // _FCH
_SPN = 80
_SPAD = _L + 16


def _make_expand():
    mesh = plsc.VectorSubcoreMesh(core_axis_name="c", subcore_axis_name="s")

    @functools.partial(
        pl.kernel,
        mesh=mesh,
        compiler_params=pltpu.CompilerParams(needs_layout_passes=False),
        out_type=[
            jax.ShapeDtypeStruct((_B * _L, _D), jnp.float32),
            jax.ShapeDtypeStruct((_B, 16), jnp.int32),
        ],
        scratch_types=[
            pltpu.VMEM((_T,), jnp.float32),
            pltpu.VMEM((_SPAD,), jnp.int32),
            pltpu.VMEM((_L // 2 + _FCH,), jnp.int32),
            pltpu.VMEM((2, _SPN, _D), jnp.float32),
            pltpu.VMEM((2, _FCH, _D), jnp.float32),
            pltpu.VMEM((16,), jnp.int32),
            pltpu.VMEM((16,), jnp.int32),
            pltpu.SemaphoreType.DMA,
            pltpu.SemaphoreType.DMA,
            pltpu.SemaphoreType.DMA,
            pltpu.SemaphoreType.DMA,
        ],
    )
    def expand(table_hbm, dur_hbm, ml_hbm, out_hbm, len_hbm,
               dur_v, s_v, idx_v, span_v, outb_v, len_v, ml_v,
               gsem_a, gsem_b, ssem_a, ssem_b):
        c = lax.axis_index("c")
        s = lax.axis_index("s")
        b = c * 8 + s // 2
        h = s % 2

        pltpu.sync_copy(dur_hbm.at[b], dur_v)
        pltpu.sync_copy(ml_hbm, ml_v)

        lanes = jnp.arange(16, dtype=jnp.int32)
        neg16 = jnp.full((16,), -1, jnp.int32)

        def init_s(i, carry):
            s_v[pl.ds(i * 16, 16)] = neg16
            return carry

        lax.fori_loop(0, _SPAD // 16, init_s, 0)

        win_base = h * (_L // 2)

        def phase_a(carries, i):
            carry_len, carry_max = carries
            d = jnp.maximum(dur_v[pl.ds(i * 16, 16)], 0.0).astype(jnp.int32)
            ends = plsc.cumsum(d) + carry_len
            starts = ends - d
            tid = i * 16 + lanes
            m = (d > 0) & (starts < _L)
            starts_c = jnp.minimum(starts, _SPAD - 16)
            plsc.store_scatter(s_v, [starts_c], tid, mask=m)
            cmx = jnp.max(jnp.where(m & (starts < win_base), tid, -1))
            return (jnp.max(ends), jnp.maximum(carry_max, cmx))

        def phase_a_body(i, carries):
            return phase_a(carries, i)

        length, carry0 = lax.fori_loop(
            0, _T // 16, phase_a_body, (jnp.int32(0), jnp.int32(-1))
        )

        len_v[...] = jnp.broadcast_to(length, (16,))

        @pl.when(h == 0)
        def _():
            pltpu.sync_copy(len_v, len_hbm.at[b])

        base = b * _T
        win0 = h * (_L // 2 // 16)

        def phase_b(i, carry):
            v = s_v[pl.ds((win0 + i) * 16, 16)]
            cm = jnp.maximum(plsc.cummax(v), carry)
            g = base + jnp.minimum(jnp.maximum(cm, 0), _T - 1)
            idx_v[pl.ds(i * 16, 16)] = g
            return jnp.max(cm)

        cm_fin = lax.fori_loop(0, _L // 2 // 16, phase_b, carry0)

        g_last = jnp.broadcast_to(
            base + jnp.minimum(jnp.maximum(cm_fin, 0), _T - 1), (16,)
        ).astype(jnp.int32)
        for q in range(_FCH // 16):
            idx_v[pl.ds(_L // 2 + q * 16, 16)] = g_last

        mls = jnp.max(ml_v[...])
        len_eff = jnp.minimum(length, mls)
        nvalid = jnp.minimum(jnp.maximum(len_eff - win_base, 0), _L // 2)
        out_base = b * _L + win_base
        cap = _B * _T - _SPN
        zf = jnp.zeros((16,), jnp.float32)

        def chunk_info(k):
            first_g = idx_v[pl.ds(k * _FCH, 16)]
            last_g = idx_v[pl.ds(k * _FCH + _FCH - 16, 16)]
            row0 = jnp.minimum((first_g[0] // 8) * 8, cap)
            ok = (last_g[15] - row0) <= (_SPN - 1)
            return row0, ok

        def read_issue(k, par, row0, ok, sem):
            @pl.when(ok)
            def _():
                pltpu.async_copy(
                    table_hbm.at[pl.ds(row0, _SPN)], span_v.at[par], sem
                )

            @pl.when(jnp.logical_not(ok))
            def _():
                pltpu.async_copy(
                    table_hbm.at[idx_v.at[pl.ds(k * _FCH, _SPN)]],
                    span_v.at[par], sem,
                )

        def read_wait(par, sem):
            pltpu.make_async_copy(
                table_hbm.at[pl.ds(0, _SPN)], span_v.at[par], sem
            ).wait()

        def scat_issue(k, par, sem):
            pltpu.async_copy(
                outb_v.at[par],
                out_hbm.at[pl.ds(out_base + k * _FCH, _FCH)], sem,
            )

        def scat_drain(par, sem):
            pltpu.make_async_copy(
                outb_v.at[par], out_hbm.at[pl.ds(out_base, _FCH)], sem
            ).wait()

        def expand_chunk(k, par, row0, ok):
            span_ref = span_v.at[par]
            outb_ref = outb_v.at[par]

            @plsc.parallel_loop(0, _FCH // 16)
            def group(gi):
                idxg = idx_v[pl.ds(k * _FCH + gi * 16, 16)]
                loff = jnp.where(ok, idxg - row0, gi * 16 + lanes)
                for lane in range(16):
                    lr = loff[lane]
                    row = gi * 16 + lane
                    for jj in range(_D // 16):
                        outb_ref[row, pl.ds(jj * 16, 16)] = span_ref[
                            lr, pl.ds(jj * 16, 16)
                        ]

            nv = jnp.minimum(jnp.maximum(nvalid - k * _FCH, 0), _FCH)

            @plsc.parallel_loop(nv, _FCH)
            def zrow(r):
                for jj in range(_D // 16):
                    outb_ref[r, pl.ds(jj * 16, 16)] = zf

        row0_0, ok_0 = chunk_info(jnp.int32(0))
        read_issue(jnp.int32(0), 0, row0_0, ok_0, gsem_a)

        def outer(j, carry):
            row0_e, ok_e = carry
            k0 = 2 * j
            k1 = 2 * j + 1

            @pl.when(j >= 1)
            def _():
                scat_drain(0, ssem_a)

            row0_o, ok_o = chunk_info(k1)
            read_issue(k1, 1, row0_o, ok_o, gsem_b)

            read_wait(0, gsem_a)
            expand_chunk(k0, 0, row0_e, ok_e)
            scat_issue(k0, 0, ssem_a)

            @pl.when(j >= 1)
            def _():
                scat_drain(1, ssem_b)

            knext = jnp.minimum(k0 + 2, _NC - 1)
            row0_n, ok_n = chunk_info(knext)

            @pl.when(j < _NC // 2 - 1)
            def _():
                read_issue(knext, 0, row0_n, ok_n, gsem_a)

            read_wait(1, gsem_b)
            expand_chunk(k1, 1, row0_o, ok_o)
            scat_issue(k1, 1, ssem_b)
            return (row0_n, ok_n)

        lax.fori_loop(0, _NC // 2, outer, (row0_0, ok_0))
        scat_drain(0, ssem_a)
        scat_drain(1, ssem_b)

    return expand


_EXPAND = _make_expand()


def kernel(hidden, durations, max_len):
    B, T, D = hidden.shape
    table = hidden.reshape(B * T, D)
    ml = jnp.minimum(jnp.asarray(max_len, jnp.int32), _L)
    mlv = jnp.broadcast_to(ml, (16,))
    out2d, len2d = _EXPAND(table, durations, mlv)
    return out2d.reshape(B, _L, D), len2d[:, 0]


# speedup vs baseline: 1.4856x; 1.2114x over previous
import functools

import jax
import jax.numpy as jnp
from jax import lax
from jax.experimental import pallas as pl
from jax.experimental.pallas import tpu as pltpu
from jax.experimental.pallas import tpu_sc as plsc

_B, _T, _D = 16, 512, 384
_L = 2048
_RCH = 128
_NCH = (_L // 2) // _RCH
_SPAD = _L + 16


def _make_expand():
    mesh = plsc.VectorSubcoreMesh(core_axis_name="c", subcore_axis_name="s")

    @functools.partial(
        pl.kernel,
        mesh=mesh,
        compiler_params=pltpu.CompilerParams(needs_layout_passes=False),
        out_type=[
            jax.ShapeDtypeStruct((_B * _L, _D), jnp.float32),
            jax.ShapeDtypeStruct((_B, 16), jnp.int32),
        ],
        scratch_types=[
            pltpu.VMEM((_T,), jnp.float32),
            pltpu.VMEM((_SPAD,), jnp.int32),
            pltpu.VMEM((_L // 2,), jnp.int32),
            pltpu.VMEM((2, _RCH, _D), jnp.float32),
            pltpu.VMEM((16,), jnp.int32),
            pltpu.VMEM((16,), jnp.int32),
            pltpu.SemaphoreType.DMA,
            pltpu.SemaphoreType.DMA,
            pltpu.SemaphoreType.DMA,
        ],
    )
    def expand(table_hbm, dur_hbm, ml_hbm, out_hbm, len_hbm,
               dur_v, s_v, idx_v, rows_v, len_v, ml_v, gsem, gsem2, ssem):
        c = lax.axis_index("c")
        s = lax.axis_index("s")
        b = c * 8 + s // 2
        h = s % 2

        pltpu.sync_copy(dur_hbm.at[b], dur_v)
        pltpu.sync_copy(ml_hbm, ml_v)

        lanes = jnp.arange(16, dtype=jnp.int32)
        neg16 = jnp.full((16,), -1, jnp.int32)

        def init_s(i, carry):
            s_v[pl.ds(i * 16, 16)] = neg16
            return carry

        lax.fori_loop(0, _SPAD // 16, init_s, 0)

        win_base = h * (_L // 2)

        def phase_a(carries, i):
            carry_len, carry_max = carries
            d = jnp.maximum(dur_v[pl.ds(i * 16, 16)], 0.0).astype(jnp.int32)
            ends = plsc.cumsum(d) + carry_len
            starts = ends - d
            tid = i * 16 + lanes
            m = (d > 0) & (starts < _L)
            starts_c = jnp.minimum(starts, _SPAD - 16)
            plsc.store_scatter(s_v, [starts_c], tid, mask=m)
            cmx = jnp.max(jnp.where(m & (starts < win_base), tid, -1))
            return (jnp.max(ends), jnp.maximum(carry_max, cmx))

        def phase_a_body(i, carries):
            return phase_a(carries, i)

        length, carry0 = lax.fori_loop(
            0, _T // 16, phase_a_body, (jnp.int32(0), jnp.int32(-1))
        )

        len_v[...] = jnp.broadcast_to(length, (16,))

        @pl.when(h == 0)
        def _():
            pltpu.sync_copy(len_v, len_hbm.at[b])

        base = b * _T
        win0 = h * (_L // 2 // 16)

        def phase_b(i, carry):
            v = s_v[pl.ds((win0 + i) * 16, 16)]
            cm = jnp.maximum(plsc.cummax(v), carry)
            g = base + jnp.minimum(jnp.maximum(cm, 0), _T - 1)
            idx_v[pl.ds(i * 16, 16)] = g
            return jnp.max(cm)

        lax.fori_loop(0, _L // 2 // 16, phase_b, carry0)

        mls = jnp.max(ml_v[...])
        len_eff = jnp.minimum(length, mls)
        nvalid = jnp.minimum(jnp.maximum(len_eff - win_base, 0), _L // 2)
        out_base = b * _L + win_base
        zf = jnp.zeros((16,), jnp.float32)

        def gather_issue(k):
            half = _RCH // 2
            d1 = pltpu.async_copy(
                table_hbm.at[idx_v.at[pl.ds(k * _RCH, half)]],
                rows_v.at[k % 2, pl.ds(0, half)], gsem,
            )
            d2 = pltpu.async_copy(
                table_hbm.at[idx_v.at[pl.ds(k * _RCH + half, half)]],
                rows_v.at[k % 2, pl.ds(half, half)], gsem2,
            )
            return (d1, d2)

        g_descs = [None] * _NCH
        s_descs = [None] * _NCH
        g_descs[0] = gather_issue(0)
        for kch in range(_NCH):
            buf = rows_v.at[kch % 2]
            if kch + 1 < _NCH:
                if kch >= 1:
                    s_descs[kch - 1].wait()
                g_descs[kch + 1] = gather_issue(kch + 1)
            g_descs[kch][0].wait()
            g_descs[kch][1].wait()
            nv = jnp.minimum(jnp.maximum(nvalid - kch * _RCH, 0), _RCH)

            def zero_row(r, carry, buf=buf):
                for j in range(_D // 16):
                    buf[r, pl.ds(j * 16, 16)] = zf
                return carry

            lax.fori_loop(nv, _RCH, zero_row, 0)
            s_descs[kch] = pltpu.async_copy(
                buf, out_hbm.at[pl.ds(out_base + kch * _RCH, _RCH)], ssem
            )
        s_descs[_NCH - 2].wait()
        s_descs[_NCH - 1].wait()

    return expand


_EXPAND = _make_expand()


def kernel(hidden, durations, max_len):
    B, T, D = hidden.shape
    table = hidden.reshape(B * T, D)
    ml = jnp.minimum(jnp.asarray(max_len, jnp.int32), _L)
    mlv = jnp.broadcast_to(ml, (16,))
    out2d, len2d = _EXPAND(table, durations, mlv)
    return out2d.reshape(B, _L, D), len2d[:, 0]


# speedup vs baseline: 1.4900x; 1.0030x over previous
import functools

import jax
import jax.numpy as jnp
from jax import lax
from jax.experimental import pallas as pl
from jax.experimental.pallas import tpu as pltpu
from jax.experimental.pallas import tpu_sc as plsc

_B, _T, _D = 16, 512, 384
_L = 2048
_RCH = 128
_NCH = (_L // 2) // _RCH
_SPAD = _L + 16


def _make_expand():
    mesh = plsc.VectorSubcoreMesh(core_axis_name="c", subcore_axis_name="s")

    @functools.partial(
        pl.kernel,
        mesh=mesh,
        compiler_params=pltpu.CompilerParams(needs_layout_passes=False),
        out_type=[
            jax.ShapeDtypeStruct((_B * _L, _D), jnp.float32),
            jax.ShapeDtypeStruct((_B, 16), jnp.int32),
        ],
        scratch_types=[
            pltpu.VMEM((_T,), jnp.float32),
            pltpu.VMEM((_SPAD,), jnp.int32),
            pltpu.VMEM((_L // 2,), jnp.int32),
            pltpu.VMEM((2, _RCH, _D), jnp.float32),
            pltpu.VMEM((16,), jnp.int32),
            pltpu.VMEM((16,), jnp.int32),
            pltpu.SemaphoreType.DMA,
            pltpu.SemaphoreType.DMA,
        ],
    )
    def expand(table_hbm, dur_hbm, ml_hbm, out_hbm, len_hbm,
               dur_v, s_v, idx_v, rows_v, len_v, ml_v, gsem, ssem):
        c = lax.axis_index("c")
        s = lax.axis_index("s")
        b = c * 8 + s // 2
        h = s % 2

        pltpu.sync_copy(dur_hbm.at[b], dur_v)
        pltpu.sync_copy(ml_hbm, ml_v)

        lanes = jnp.arange(16, dtype=jnp.int32)
        neg16 = jnp.full((16,), -1, jnp.int32)

        def init_s(i, carry):
            s_v[pl.ds(i * 16, 16)] = neg16
            return carry

        lax.fori_loop(0, _SPAD // 16, init_s, 0)

        win_base = h * (_L // 2)

        def phase_a(carries, i):
            carry_len, carry_max = carries
            d = jnp.maximum(dur_v[pl.ds(i * 16, 16)], 0.0).astype(jnp.int32)
            ends = plsc.cumsum(d) + carry_len
            starts = ends - d
            tid = i * 16 + lanes
            m = (d > 0) & (starts < _L)
            starts_c = jnp.minimum(starts, _SPAD - 16)
            plsc.store_scatter(s_v, [starts_c], tid, mask=m)
            cmx = jnp.max(jnp.where(m & (starts < win_base), tid, -1))
            return (jnp.max(ends), jnp.maximum(carry_max, cmx))

        def phase_a_body(i, carries):
            return phase_a(carries, i)

        length, carry0 = lax.fori_loop(
            0, _T // 16, phase_a_body, (jnp.int32(0), jnp.int32(-1))
        )

        len_v[...] = jnp.broadcast_to(length, (16,))

        @pl.when(h == 0)
        def _():
            pltpu.sync_copy(len_v, len_hbm.at[b])

        base = b * _T
        win0 = h * (_L // 2 // 16)

        def phase_b(i, carry):
            v = s_v[pl.ds((win0 + i) * 16, 16)]
            cm = jnp.maximum(plsc.cummax(v), carry)
            g = base + jnp.minimum(jnp.maximum(cm, 0), _T - 1)
            idx_v[pl.ds(i * 16, 16)] = g
            return jnp.max(cm)

        lax.fori_loop(0, _L // 2 // 16, phase_b, carry0)

        mls = jnp.max(ml_v[...])
        len_eff = jnp.minimum(length, mls)
        nvalid = jnp.minimum(jnp.maximum(len_eff - win_base, 0), _L // 2)
        out_base = b * _L + win_base
        zf = jnp.zeros((16,), jnp.float32)

        def gather_issue(k):
            return pltpu.async_copy(
                table_hbm.at[idx_v.at[pl.ds(k * _RCH, _RCH)]],
                rows_v.at[k % 2], gsem,
            )

        g_descs = [None] * _NCH
        s_descs = [None] * _NCH
        g_descs[0] = gather_issue(0)
        for kch in range(_NCH):
            buf = rows_v.at[kch % 2]
            if kch + 1 < _NCH:
                if kch >= 1:
                    s_descs[kch - 1].wait()
                g_descs[kch + 1] = gather_issue(kch + 1)
            g_descs[kch].wait()
            nv = jnp.minimum(jnp.maximum(nvalid - kch * _RCH, 0), _RCH)

            def zero_row(r, carry, buf=buf):
                for j in range(_D // 16):
                    buf[r, pl.ds(j * 16, 16)] = zf
                return carry

            lax.fori_loop(nv, _RCH, zero_row, 0)
            s_descs[kch] = pltpu.async_copy(
                buf, out_hbm.at[pl.ds(out_base + kch * _RCH, _RCH)], ssem
            )
        s_descs[_NCH - 2].wait()
        s_descs[_NCH - 1].wait()

    return expand


_EXPAND = _make_expand()


def kernel(hidden, durations, max_len):
    B, T, D = hidden.shape
    table = hidden.reshape(B * T, D)
    ml = jnp.minimum(jnp.asarray(max_len, jnp.int32), _L)
    mlv = jnp.broadcast_to(ml, (16,))
    out2d, len2d = _EXPAND(table, durations, mlv)
    return out2d.reshape(B, _L, D), len2d[:, 0]
